# baseline (speedup 1.0000x reference)
import math

import jax
import jax.numpy as jnp
from jax import lax
from jax.experimental import pallas as pl
from jax.experimental.pallas import tpu as pltpu
from jax.experimental.pallas import tpu_sc as plsc

D_MODEL = 512
SCALE = math.sqrt(D_MODEL)
LANES = 16

NUM_CORES = 2
NUM_SUBCORES = 16
NW = NUM_CORES * NUM_SUBCORES

BATCH = 4096
SEQ = 50
B_PER_W = BATCH // NW
CHUNK = 64
SPLITS = B_PER_W // CHUNK
NCHUNK = SEQ * SPLITS
NBUF = 3
FRONT = NBUF - 1
MAIN_ROUNDS = (NCHUNK - FRONT) // NBUF
BACK = NCHUNK - FRONT - MAIN_ROUNDS * NBUF
assert BACK >= 1


def _sc_body(table_hbm, idx_hbm, out_hbm, idx_v, *rest):
  cid = lax.axis_index("c")
  sid = lax.axis_index("s")
  wid = sid * NUM_CORES + cid

  bufs = rest[:NBUF]
  gsems = rest[NBUF:2 * NBUF]
  osems = rest[2 * NBUF:]

  pltpu.sync_copy(idx_hbm.at[wid], idx_v)

  def gcp(c, b):
    return pltpu.make_async_copy(table_hbm.at[idx_v.at[c]], bufs[b], gsems[b])

  def ocp(c, b):
    s = c // SPLITS
    h = c % SPLITS
    return pltpu.make_async_copy(
        bufs[b], out_hbm.at[s, pl.ds(wid * B_PER_W + h * CHUNK, CHUNK)],
        osems[b])

  def scale_buf(b):
    buf = bufs[b]

    @plsc.parallel_loop(0, CHUNK)
    def _(r):
      for j in range(D_MODEL // LANES):
        sl = pl.ds(j * LANES, LANES)
        buf[r, sl] = buf[r, sl] * SCALE

  gcp(0, 0).start()

  def do_chunk(g, b, has_prev_wait, start_next):
    nb = (b + 1) % NBUF
    gcp(g, b).wait()
    scale_buf(b)
    ocp(g, b).start()
    if start_next:
      if has_prev_wait:
        ocp(g + 1 - NBUF, nb).wait()
      gcp(g + 1, nb).start()

  for g in range(FRONT):
    do_chunk(g, g % NBUF, False, True)

  def loop_body(p, carry):
    for b in range(NBUF):
      g = FRONT + p * NBUF + b
      do_chunk(g, (FRONT + b) % NBUF, True, True)
    return carry

  lax.fori_loop(0, MAIN_ROUNDS, loop_body, jnp.int32(0))

  for k in range(BACK):
    g = FRONT + MAIN_ROUNDS * NBUF + k
    do_chunk(g, g % NBUF, True, g + 1 < NCHUNK)

  for g in range(NCHUNK - NBUF, NCHUNK):
    ocp(g, g % NBUF).wait()


def _make_sc_call():
  mesh = plsc.VectorSubcoreMesh(core_axis_name="c", subcore_axis_name="s")
  return pl.kernel(
      _sc_body,
      out_type=jax.ShapeDtypeStruct((SEQ, BATCH, D_MODEL), jnp.float32),
      mesh=mesh,
      scratch_types=(
          [pltpu.VMEM((NCHUNK, CHUNK), jnp.int32)]
          + [pltpu.VMEM((CHUNK, D_MODEL), jnp.float32)] * NBUF
          + [pltpu.SemaphoreType.DMA] * (2 * NBUF)
      ),
      name="embedding_gather_scale_sc",
  )


def kernel(x, table):
  idx = (
      x.astype(jnp.int32)
      .T.reshape(SEQ, NW, SPLITS, CHUNK)
      .transpose(1, 0, 2, 3)
      .reshape(NW, NCHUNK, CHUNK)
  )
  out = _make_sc_call()(table, idx)
  return out.transpose(1, 0, 2)


# speedup vs baseline: 1.3375x; 1.3375x over previous
import math

import jax
import jax.numpy as jnp
from jax import lax
from jax.experimental import pallas as pl
from jax.experimental.pallas import tpu as pltpu
from jax.experimental.pallas import tpu_sc as plsc

D_MODEL = 512
SCALE = math.sqrt(D_MODEL)
LANES = 16

NUM_CORES = 2
NUM_SUBCORES = 16
NW = NUM_CORES * NUM_SUBCORES

BATCH = 4096
SEQ = 50
B_PER_W = BATCH // NW
CHUNK = 64
SPLITS = B_PER_W // CHUNK
NCHUNK = SEQ * SPLITS
NBUF = 2


def _sc_body(table_hbm, idx_hbm, out_hbm, idx_v, *rest):
  cid = lax.axis_index("c")
  sid = lax.axis_index("s")
  wid = sid * NUM_CORES + cid

  bufs = rest[:NBUF]
  gsems = rest[NBUF:2 * NBUF]
  osems = rest[2 * NBUF:]

  pltpu.sync_copy(idx_hbm.at[wid], idx_v)

  def gcp(c, b):
    return pltpu.make_async_copy(table_hbm.at[idx_v.at[c]], bufs[b], gsems[b])

  def ocp(c, b):
    s = c // SPLITS
    h = c % SPLITS
    return pltpu.make_async_copy(
        bufs[b], out_hbm.at[s, pl.ds(wid * B_PER_W + h * CHUNK, CHUNK)],
        osems[b])

  def scale_buf(b):
    buf = bufs[b]

    @plsc.parallel_loop(0, CHUNK)
    def _(r):
      for j in range(D_MODEL // LANES):
        sl = pl.ds(j * LANES, LANES)
        buf[r, sl] = buf[r, sl] * SCALE

  for b in range(NBUF):
    gcp(b, b).start()

  def do_round(p, start_next):
    for b in range(NBUF):
      g = p * NBUF + b
      gcp(g, b).wait()
      scale_buf(b)
      ocp(g, b).start()
      if start_next:
        ocp(g, b).wait()
        gcp(g + NBUF, b).start()

  nrounds = NCHUNK // NBUF

  def loop_body(p, carry):
    do_round(p, True)
    return carry

  lax.fori_loop(0, nrounds - 1, loop_body, jnp.int32(0))
  do_round(nrounds - 1, False)

  for b in range(NBUF):
    ocp(NCHUNK - NBUF + b, b).wait()


def _make_sc_call():
  mesh = plsc.VectorSubcoreMesh(core_axis_name="c", subcore_axis_name="s")
  return pl.kernel(
      _sc_body,
      out_type=jax.ShapeDtypeStruct((SEQ, BATCH, D_MODEL), jnp.float32),
      mesh=mesh,
      scratch_types=(
          [pltpu.VMEM((NCHUNK, CHUNK), jnp.int32)]
          + [pltpu.VMEM((CHUNK, D_MODEL), jnp.float32)] * NBUF
          + [pltpu.SemaphoreType.DMA] * (2 * NBUF)
      ),
      name="embedding_gather_scale_sc",
  )


def kernel(x, table):
  idx = (
      x.astype(jnp.int32)
      .T.reshape(SEQ, NW, SPLITS, CHUNK)
      .transpose(1, 0, 2, 3)
      .reshape(NW, NCHUNK, CHUNK)
  )
  out = _make_sc_call()(table, idx)
  return out.transpose(1, 0, 2)
